# baseline (speedup 1.0000x reference)
import functools

import jax
import jax.numpy as jnp
from jax import lax
from jax.experimental import pallas as pl
from jax.experimental.pallas import tpu as pltpu
from jax.experimental.pallas import tpu_sc as plsc

B = 16384
EMB = 64
W = 128
NC = 2
NS = 16
NW = NC * NS
BPW = B // NW
CHUNK = 64
NCHUNK = BPW // CHUNK
NT = 7


def _sc_kernel(*refs):
    idx_hbm = refs[0:NT]
    par_hbm = refs[NT:2 * NT]
    tables = refs[2 * NT:3 * NT]
    out = refs[3 * NT]
    scratch = refs[3 * NT + 1:]
    idx_v = scratch[0:NT]
    par_v = scratch[NT:2 * NT]
    rows_v = scratch[2 * NT:4 * NT]
    out_v = scratch[4 * NT]
    tr_v = scratch[4 * NT + 1]
    sems = scratch[4 * NT + 2:]

    wid = lax.axis_index("s") * NC + lax.axis_index("c")
    base = wid * BPW

    for t in range(NT):
        pltpu.sync_copy(idx_hbm[t].at[pl.ds(base, BPW)], idx_v[t])
        pltpu.sync_copy(par_hbm[t].at[pl.ds(base, BPW)], par_v[t])

    def gathers(c, slot):
        return [pltpu.make_async_copy(
            tables[t].at[idx_v[t].at[pl.ds(c * CHUNK, CHUNK)]],
            rows_v[slot * NT + t], sems[slot]) for t in range(NT)]

    def start(c, slot):
        for cp in gathers(c, slot):
            cp.start()

    def wait(c, slot):
        for cp in gathers(c, slot):
            cp.wait()

    def compute(c, slot):
        bufs = rows_v[slot * NT:slot * NT + NT]
        iota = lax.iota(jnp.int32, 16)

        def group(g, _):
            rbase = c * CHUNK + g * 16
            pvs = [par_v[t][pl.ds(rbase, 16)] for t in range(NT)]
            for r in range(16):
                row16 = jnp.full((16,), 0, jnp.int32) + (g * 16 + r)
                lane_r = jnp.reshape(iota * 0 + r, (16, 1))
                dnums = lax.GatherDimensionNumbers(
                    offset_dims=(), collapsed_slice_dims=(0,),
                    start_index_map=(0,))
                cols = [lax.gather(
                    pvs[t], lane_r, dnums, slice_sizes=(1,),
                    mode=lax.GatherScatterMode.PROMISE_IN_BOUNDS) + iota
                        for t in range(NT)]
                s = None
                for q in range(4):
                    p = None
                    for t in range(NT):
                        f = plsc.load_gather(bufs[t], [row16, cols[t] + q * 16])
                        p = f if p is None else p * f
                    s = p if s is None else s + p
                tr_v[pl.ds(r * 17, 16)] = s
            acc = jnp.zeros((16,), jnp.float32)
            tcols = iota * 17
            for l in range(16):
                acc = acc + plsc.load_gather(tr_v, [tcols + l])
            out_v[pl.ds(rbase, 16)] = acc
            return 0

        lax.fori_loop(0, CHUNK // 16, group, 0)

    start(0, 0)
    start(1, 1)

    def chunk_pair(cp_i, _):
        c = cp_i * 2
        wait(c, 0)
        compute(c, 0)

        @pl.when(cp_i + 1 < NCHUNK // 2)
        def _():
            start(c + 2, 0)
        wait(c + 1, 1)
        compute(c + 1, 1)

        @pl.when(cp_i + 1 < NCHUNK // 2)
        def _():
            start(c + 3, 1)
        return 0

    lax.fori_loop(0, NCHUNK // 2, chunk_pair, 0)

    pltpu.sync_copy(out_v, out.at[pl.ds(base, BPW)])


def kernel(r_idx, e1_idx, e2_idx, e3_idx, e4_idx, e5_idx, e6_idx,
           E1, E2, E3, E4, E5, E6, R):
    idxs = [jnp.asarray(i, jnp.int32)
            for i in (r_idx, e1_idx, e2_idx, e3_idx, e4_idx, e5_idx, e6_idx)]
    widx = [i >> 1 for i in idxs]
    par = [(i & 1) * 64 for i in idxs]
    tabs = [jnp.reshape(T, (T.shape[0] // 2, 2 * EMB))
            for T in (R, E1, E2, E3, E4, E5, E6)]
    mesh = plsc.VectorSubcoreMesh(core_axis_name="c", subcore_axis_name="s")
    scratch = (
        [pltpu.VMEM((BPW,), jnp.int32) for _ in range(2 * NT)]
        + [pltpu.VMEM((CHUNK, W), jnp.float32) for _ in range(2 * NT)]
        + [pltpu.VMEM((BPW,), jnp.float32)]
        + [pltpu.VMEM((16 * 17,), jnp.float32)]
        + [pltpu.SemaphoreType.DMA, pltpu.SemaphoreType.DMA]
    )
    f = functools.partial(
        pl.kernel, mesh=mesh,
        out_type=jax.ShapeDtypeStruct((B,), jnp.float32),
        scratch_types=scratch,
        compiler_params=pltpu.CompilerParams(
            needs_layout_passes=False, use_tc_tiling_on_sc=True),
    )(_sc_kernel)
    return f(*widx, *par, *tabs)
